# Initial kernel scaffold; baseline (speedup 1.0000x reference)
import jax
import jax.numpy as jnp
from jax.experimental import pallas as pl


def kernel(x, edge_index, edge_attr, batch, Wl, bl, Wr, br, We, att, bg, Wsl, bsl, Wsr, gn_w, gn_b, gn_ms, W1, b1, W2, b2, W3, b3):
    raise NotImplementedError("write your pallas kernel here")


# speedup vs baseline: 23.7655x; 23.7655x over previous
import functools

import jax
import jax.numpy as jnp
from jax import lax
from jax.experimental import pallas as pl
from jax.experimental.pallas import tpu as pltpu
from jax.experimental.pallas import tpu_sc as plsc

N = 50000
E = 800000
G = 16
H = 16
C = 4
HC = H * C
HID = 64

NW = 32
N_P = 50176
E_P = 819200
EPW = E_P // NW
K = 128
NIT = EPW // K
SAGE_W = 32
DEG_W = 8
HALF = N_P // 2
HR = HALF + 128
RPH = HR // 16

BLK = 512
NB = N_P // BLK

_mesh = plsc.VectorSubcoreMesh(core_axis_name="c", subcore_axis_name="s")
_sc_params = pltpu.CompilerParams(needs_layout_passes=False,
                                  use_tc_tiling_on_sc=False)


def _zero_acc(zeros_hbm, acc_sh, sid):
    r0 = pl.multiple_of(sid * RPH, 8)
    pltpu.sync_copy(zeros_hbm, acc_sh.at[pl.ds(r0, RPH)])


@functools.partial(
    pl.kernel,
    out_type=(
        jax.ShapeDtypeStruct((E_P, H), jnp.float32),
        jax.ShapeDtypeStruct((E_P, H), jnp.float32),
    ),
    mesh=_mesh,
    compiler_params=_sc_params,
    scratch_types=[
        pltpu.VMEM((N_P,), jnp.float32),
        pltpu.VMEM((K,), jnp.int32),
        pltpu.VMEM((K,), jnp.int32),
        pltpu.VMEM((K,), jnp.float32),
        pltpu.VMEM((K, H), jnp.float32),
        pltpu.VMEM((K, H), jnp.float32),
        pltpu.VMEM((5 * C * 16,), jnp.float32),
    ],
)
def _gat_edge_sc(x_hbm, src_hbm, dst_hbm, ea_hbm, wp_hbm, ow_hbm, owx_hbm,
                 x_v, src_v, dst_v, ea_v, payw_v, paywx_v, wp_v):
    cid = lax.axis_index("c")
    sid = lax.axis_index("s")
    wid = sid * 2 + cid

    pltpu.sync_copy(wp_hbm, wp_v)
    pltpu.sync_copy(x_hbm, x_v)

    wv = [[wp_v[pl.ds((j * C + c) * 16, 16)] for c in range(C)]
          for j in range(5)]

    base = wid * EPW

    @pl.loop(0, NIT)
    def _chunk(it):
        off = pl.multiple_of(base + it * K, 8)
        pltpu.sync_copy(src_hbm.at[pl.ds(off, K)], src_v)
        pltpu.sync_copy(dst_hbm.at[pl.ds(off, K)], dst_v)
        pltpu.sync_copy(ea_hbm.at[pl.ds(off, K)], ea_v)

        @pl.loop(0, K // 16)
        def _group(g):
            g16 = g * 16
            s16 = src_v[pl.ds(g16, 16)]
            d16 = dst_v[pl.ds(g16, 16)]
            e16 = ea_v[pl.ds(g16, 16)]
            xs16 = plsc.load_gather(x_v, [s16])
            xd16 = plsc.load_gather(x_v, [d16])
            for l in range(16):
                xs = xs16[l]
                xd = xd16[l]
                ea = e16[l]
                lg = None
                for c in range(C):
                    t = (xs * wv[0][c] + xd * wv[1][c]
                         + ea * wv[2][c] + wv[3][c])
                    m = jnp.maximum(t, 0.2 * t)
                    contrib = m * wv[4][c]
                    lg = contrib if lg is None else lg + contrib
                w = jnp.exp(lg)
                r = g16 + l
                payw_v[r] = w
                paywx_v[r] = w * xs

        pltpu.sync_copy(payw_v, ow_hbm.at[pl.ds(off, K)])
        pltpu.sync_copy(paywx_v, owx_hbm.at[pl.ds(off, K)])


@functools.partial(
    pl.kernel,
    out_type=(
        jax.ShapeDtypeStruct((2, HR, H), jnp.float32),
        jax.ShapeDtypeStruct((2, HR, H), jnp.float32),
        jax.ShapeDtypeStruct((2, HR, DEG_W), jnp.float32),
    ),
    mesh=_mesh,
    compiler_params=_sc_params,
    scratch_types=[
        pltpu.VMEM((K,), jnp.int32),
        pltpu.VMEM((K, H), jnp.float32),
        pltpu.VMEM((K, H), jnp.float32),
        pltpu.VMEM((K, DEG_W), jnp.float32),
        pltpu.VMEM_SHARED((HR, H), jnp.float32),
        pltpu.VMEM_SHARED((HR, H), jnp.float32),
        pltpu.VMEM_SHARED((HR, DEG_W), jnp.float32),
    ],
)
def _scat_sc(pw_hbm, pwx_hbm, idx_hbm, ones_hbm, z16_hbm, z8_hbm,
             ow_hbm, owx_hbm, od_hbm,
             idx_v, payw_v, paywx_v, payd_v,
             accw_sh, accwx_sh, accd_sh):
    cid = lax.axis_index("c")
    sid = lax.axis_index("s")
    wid = sid * 2 + cid

    pltpu.sync_copy(ones_hbm, payd_v)
    _zero_acc(z16_hbm, accw_sh, sid)
    _zero_acc(z16_hbm, accwx_sh, sid)
    _zero_acc(z8_hbm, accd_sh, sid)
    plsc.subcore_barrier()

    base = wid * EPW

    @pl.loop(0, NIT)
    def _chunk(it):
        off = pl.multiple_of(base + it * K, 8)
        pltpu.sync_copy(idx_hbm.at[pl.ds(off, K)], idx_v)
        pltpu.sync_copy(pw_hbm.at[pl.ds(off, K)], payw_v)
        pltpu.sync_copy(pwx_hbm.at[pl.ds(off, K)], paywx_v)
        pltpu.sync_copy(payw_v, accw_sh.at[idx_v], add=True)
        pltpu.sync_copy(paywx_v, accwx_sh.at[idx_v], add=True)
        pltpu.sync_copy(payd_v, accd_sh.at[idx_v], add=True)

    plsc.subcore_barrier()
    r0 = pl.multiple_of(sid * RPH, 8)
    sl = pl.ds(r0, RPH)
    pltpu.sync_copy(accw_sh.at[sl], ow_hbm.at[cid, sl])
    pltpu.sync_copy(accwx_sh.at[sl], owx_hbm.at[cid, sl])
    pltpu.sync_copy(accd_sh.at[sl], od_hbm.at[cid, sl])


@functools.partial(
    pl.kernel,
    out_type=jax.ShapeDtypeStruct((2, HR, SAGE_W), jnp.float32),
    mesh=_mesh,
    compiler_params=_sc_params,
    scratch_types=[
        pltpu.VMEM((K,), jnp.int32),
        pltpu.VMEM((K,), jnp.int32),
        pltpu.VMEM((K, SAGE_W), jnp.float32),
        pltpu.VMEM_SHARED((HR, SAGE_W), jnp.float32),
        pltpu.SemaphoreType.DMA,
    ],
)
def _sage_sc(h1_hbm, src_hbm, idx_hbm, z32_hbm, out_hbm,
             src_v, idx_v, rows_v, acc_sh, sem):
    cid = lax.axis_index("c")
    sid = lax.axis_index("s")
    wid = sid * 2 + cid

    _zero_acc(z32_hbm, acc_sh, sid)
    plsc.subcore_barrier()

    base = wid * EPW

    @pl.loop(0, NIT)
    def _chunk(it):
        off = pl.multiple_of(base + it * K, 8)
        pltpu.sync_copy(src_hbm.at[pl.ds(off, K)], src_v)
        pltpu.sync_copy(idx_hbm.at[pl.ds(off, K)], idx_v)
        pltpu.async_copy(h1_hbm.at[src_v], rows_v, sem).wait()
        pltpu.sync_copy(rows_v, acc_sh.at[idx_v], add=True)

    plsc.subcore_barrier()
    r0 = pl.multiple_of(sid * RPH, 8)
    sl = pl.ds(r0, RPH)
    pltpu.sync_copy(acc_sh.at[sl], out_hbm.at[cid, sl])


def _easum_body(ea_ref, out_ref):
    out_ref[...] = jnp.sum(ea_ref[...]).reshape(1, 1)


def _idx_body(dst_ref, i0_ref, i1_ref):
    d = dst_ref[...]
    i0_ref[...] = jnp.where(d < HALF, d, HALF)
    i1_ref[...] = jnp.where(d >= HALF, d - HALF, HALF)


def _onehot(batch_col):
    io = lax.broadcasted_iota(jnp.int32, (1, G), 1)
    return (batch_col == io).astype(jnp.float32)


def _b1_body(accw_ref, accwx_ref, accd_ref, x_ref, b_ref, wsum_ref, we_ref,
             bsum_ref, amat_ref, wl_ref, bl_ref, bg_ref, eh_ref, easum_ref,
             h1_ref, deg_ref, sums_ref, cnt_ref, sums_s, cnt_s):
    i = pl.program_id(0)

    @pl.when(i == 0)
    def _():
        sums_s[...] = jnp.zeros_like(sums_s)
        cnt_s[...] = jnp.zeros_like(cnt_s)

    den16 = accw_ref[0] + accw_ref[1]
    s116 = accwx_ref[0] + accwx_ref[1]
    deg_ref[...] = accd_ref[0][:, 0:1] + accd_ref[1][:, 0:1]

    xb = x_ref[...]
    eamean = easum_ref[0, 0] * (1.0 / E)
    t = xb * wsum_ref[...] + eamean * we_ref[...] + bsum_ref[...]
    m = jnp.maximum(t, 0.2 * t)
    logits = jnp.dot(m, amat_ref[...], preferred_element_type=jnp.float32)
    wself = jnp.exp(logits)
    den16 = den16 + wself
    s116 = s116 + wself * xb

    den64 = jnp.dot(den16, eh_ref[...], preferred_element_type=jnp.float32)
    s164 = jnp.dot(s116, eh_ref[...], preferred_element_type=jnp.float32)
    num = wl_ref[...] * s164 + bl_ref[...] * den64
    h1p = jax.nn.relu(num / (den64 + 1e-16) + bg_ref[...])
    h1_ref[...] = h1p

    oh = _onehot(b_ref[...])
    sums_s[...] += lax.dot_general(oh, h1p, (((0,), (0,)), ((), ())),
                                   preferred_element_type=jnp.float32)
    cnt_s[...] += jnp.sum(oh, axis=0, keepdims=True)

    @pl.when(i == NB - 1)
    def _():
        sums_ref[...] = sums_s[...]
        cnt_ref[...] = cnt_s[...]


def _b2_body(h1_ref, b_ref, sums_ref, cntc_ref, gnms_ref,
             cen_ref, ssq_ref, ssq_s):
    i = pl.program_id(0)

    @pl.when(i == 0)
    def _():
        ssq_s[...] = jnp.zeros_like(ssq_s)

    mean = sums_ref[...] / cntc_ref[...]
    oh = _onehot(b_ref[...])
    mb = jnp.dot(oh, mean, preferred_element_type=jnp.float32)
    cen = h1_ref[...] - gnms_ref[...] * mb
    cen_ref[...] = cen
    ssq_s[...] += lax.dot_general(oh, cen * cen, (((0,), (0,)), ((), ())),
                                  preferred_element_type=jnp.float32)

    @pl.when(i == NB - 1)
    def _():
        ssq_ref[...] = ssq_s[...]


def _b3_body(cen_ref, b_ref, ssq_ref, cntc_ref, gnw_ref, gnb_ref, wsr_ref,
             h1a_ref, h1b_ref, t2_ref):
    var = ssq_ref[...] / cntc_ref[...]
    scale = 1.0 / jnp.sqrt(var + 1e-5)
    oh = _onehot(b_ref[...])
    sb = jnp.dot(oh, scale, preferred_element_type=jnp.float32)
    h1 = gnw_ref[...] * cen_ref[...] * sb + gnb_ref[...]
    h1a_ref[...] = h1[:, :32]
    h1b_ref[...] = h1[:, 32:]
    t2_ref[...] = jnp.dot(h1, wsr_ref[...], preferred_element_type=jnp.float32)


def _c_body(aa_ref, ab_ref, deg_ref, t2_ref, b_ref, wsl_ref, bsl_ref,
            cntc_ref, gmax_ref, gmean_ref, gmax_s, gsum_s):
    i = pl.program_id(0)

    @pl.when(i == 0)
    def _():
        gmax_s[...] = jnp.full_like(gmax_s, -jnp.inf)
        gsum_s[...] = jnp.zeros_like(gsum_s)

    agg = jnp.concatenate([aa_ref[0] + aa_ref[1], ab_ref[0] + ab_ref[1]],
                          axis=1)
    degc = jnp.maximum(deg_ref[...], 1.0)
    aggn = agg / degc
    h2 = jax.nn.relu(
        jnp.dot(aggn, wsl_ref[...], preferred_element_type=jnp.float32)
        + bsl_ref[...] + t2_ref[...])

    bcol = b_ref[...]
    for g in range(G):
        mask = bcol == g
        vals = jnp.where(mask, h2, -jnp.inf)
        mg = jnp.max(vals, axis=0, keepdims=True)
        gmax_s[pl.ds(g, 1), :] = jnp.maximum(gmax_s[pl.ds(g, 1), :], mg)

    oh = _onehot(bcol)
    gsum_s[...] += lax.dot_general(oh, h2, (((0,), (0,)), ((), ())),
                                   preferred_element_type=jnp.float32)

    @pl.when(i == NB - 1)
    def _():
        gm = gmax_s[...]
        gmax_ref[...] = jnp.where(jnp.isfinite(gm), gm, 0.0)
        gmean_ref[...] = gsum_s[...] / cntc_ref[...]


def _d_body(gmax_ref, gmean_ref, w1_ref, b1_ref, w2_ref, b2_ref,
            w3_ref, b3_ref, out_ref):
    z = jnp.concatenate([gmax_ref[...], gmean_ref[...]], axis=1)
    z = jax.nn.relu(jnp.dot(z, w1_ref[...], preferred_element_type=jnp.float32)
                    + b1_ref[...])
    z = jax.nn.relu(jnp.dot(z, w2_ref[...], preferred_element_type=jnp.float32)
                    + b2_ref[...])
    out_ref[...] = (jnp.dot(z, w3_ref[...], preferred_element_type=jnp.float32)
                    + b3_ref[...])


def _full(shape):
    return pl.BlockSpec(shape, lambda i: tuple(0 for _ in shape))


def _rows(width):
    return pl.BlockSpec((BLK, width), lambda i: (i, 0))


def kernel(x, edge_index, edge_attr, batch, Wl, bl, Wr, br, We, att, bg,
           Wsl, bsl, Wsr, gn_w, gn_b, gn_ms, W1, b1, W2, b2, W3, b3):
    f32 = jnp.float32
    src = edge_index[0].astype(jnp.int32)
    dst = edge_index[1].astype(jnp.int32)

    pe = E_P - E
    srcp = jnp.concatenate([src, jnp.full((pe,), N, jnp.int32)])
    dstp = jnp.concatenate([dst, jnp.full((pe,), N, jnp.int32)])
    eap = jnp.concatenate([edge_attr.astype(f32), jnp.zeros((pe,), f32)])
    xp = jnp.concatenate([x.astype(f32), jnp.zeros((N_P - N,), f32)])
    batchp = jnp.concatenate([batch.astype(jnp.int32),
                              jnp.full((N_P - N,), G, jnp.int32)])
    x2 = xp.reshape(N_P, 1)
    b2col = batchp.reshape(N_P, 1)

    wl = Wl.reshape(1, HC)
    wr = Wr.reshape(1, HC)
    we = We.reshape(1, HC)
    blr = bl.reshape(1, HC)
    bsum = (bl + br).reshape(1, HC)
    attf = att.reshape(HC)
    wpack = jnp.concatenate(
        [a.reshape(H, C).T.reshape(1, HC)
         for a in (Wl[0], Wr[0], We[0], bl + br, attf)], axis=0).reshape(-1)
    eh = jnp.kron(jnp.eye(H, dtype=f32), jnp.ones((1, C), f32))
    amat = eh.T * attf[:, None]

    easum = pl.pallas_call(
        _easum_body,
        out_shape=jax.ShapeDtypeStruct((1, 1), f32),
    )(eap.reshape(E_P // 128, 128))

    zer16 = jnp.zeros((RPH, H), f32)
    zer32 = jnp.zeros((RPH, SAGE_W), f32)
    zer8 = jnp.zeros((RPH, DEG_W), f32)
    ones8 = jnp.ones((K, DEG_W), f32)
    idx0, idx1 = pl.pallas_call(
        _idx_body,
        out_shape=[jax.ShapeDtypeStruct((E_P // 128, 128), jnp.int32)] * 2,
    )(dstp.reshape(E_P // 128, 128))
    idxs = [idx0.reshape(E_P), idx1.reshape(E_P)]
    pay_w, pay_wx = _gat_edge_sc(xp, srcp, dstp, eap, wpack)
    parts = [_scat_sc(pay_w, pay_wx, idxs[p], ones8, zer16, zer8)
             for p in (0, 1)]
    acc_w = jnp.concatenate([parts[0][0][:, :HALF], parts[1][0][:, :HALF]],
                            axis=1)
    acc_wx = jnp.concatenate([parts[0][1][:, :HALF], parts[1][1][:, :HALF]],
                             axis=1)
    acc_d = jnp.concatenate([parts[0][2][:, :HALF], parts[1][2][:, :HALF]],
                            axis=1)

    h1p, deg, sums, cnt = pl.pallas_call(
        _b1_body,
        grid=(NB,),
        in_specs=[
            pl.BlockSpec((2, BLK, H), lambda i: (0, i, 0)),
            pl.BlockSpec((2, BLK, H), lambda i: (0, i, 0)),
            pl.BlockSpec((2, BLK, DEG_W), lambda i: (0, i, 0)),
            _rows(1), _rows(1),
            _full((1, HC)), _full((1, HC)), _full((1, HC)),
            _full((HC, H)), _full((1, HC)), _full((1, HC)), _full((1, HC)),
            _full((H, HC)), _full((1, 1)),
        ],
        out_specs=[_rows(HC), _rows(1), _full((G, HC)), _full((1, G))],
        out_shape=[
            jax.ShapeDtypeStruct((N_P, HC), f32),
            jax.ShapeDtypeStruct((N_P, 1), f32),
            jax.ShapeDtypeStruct((G, HC), f32),
            jax.ShapeDtypeStruct((1, G), f32),
        ],
        scratch_shapes=[pltpu.VMEM((G, HC), f32), pltpu.VMEM((1, G), f32)],
    )(acc_w, acc_wx, acc_d, x2, b2col, wl + wr, we, bsum, amat, wl, blr,
      bg.reshape(1, HC), eh, easum)

    cntc = jnp.maximum(cnt.reshape(G, 1), 1.0)

    cen, ssq = pl.pallas_call(
        _b2_body,
        grid=(NB,),
        in_specs=[_rows(HC), _rows(1), _full((G, HC)), _full((G, 1)),
                  _full((1, HC))],
        out_specs=[_rows(HC), _full((G, HC))],
        out_shape=[jax.ShapeDtypeStruct((N_P, HC), f32),
                   jax.ShapeDtypeStruct((G, HC), f32)],
        scratch_shapes=[pltpu.VMEM((G, HC), f32)],
    )(h1p, b2col, sums, cntc, gn_ms.reshape(1, HC))

    h1a, h1b, t2 = pl.pallas_call(
        _b3_body,
        grid=(NB,),
        in_specs=[_rows(HC), _rows(1), _full((G, HC)), _full((G, 1)),
                  _full((1, HC)), _full((1, HC)), _full((HC, HID))],
        out_specs=[_rows(32), _rows(32), _rows(HID)],
        out_shape=[jax.ShapeDtypeStruct((N_P, 32), f32),
                   jax.ShapeDtypeStruct((N_P, 32), f32),
                   jax.ShapeDtypeStruct((N_P, HID), f32)],
    )(cen, b2col, ssq, cntc, gn_w.reshape(1, HC), gn_b.reshape(1, HC), Wsr)

    def _sage_full(h1half):
        ps = [_sage_sc(h1half, srcp, idxs[p], zer32) for p in (0, 1)]
        return jnp.concatenate([ps[0][:, :HALF], ps[1][:, :HALF]], axis=1)

    agg_a = _sage_full(h1a)
    agg_b = _sage_full(h1b)

    gmax, gmean = pl.pallas_call(
        _c_body,
        grid=(NB,),
        in_specs=[
            pl.BlockSpec((2, BLK, SAGE_W), lambda i: (0, i, 0)),
            pl.BlockSpec((2, BLK, SAGE_W), lambda i: (0, i, 0)),
            _rows(1), _rows(HID), _rows(1),
            _full((HID, HID)), _full((1, HID)), _full((G, 1)),
        ],
        out_specs=[_full((G, HID)), _full((G, HID))],
        out_shape=[jax.ShapeDtypeStruct((G, HID), f32),
                   jax.ShapeDtypeStruct((G, HID), f32)],
        scratch_shapes=[pltpu.VMEM((G, HID), f32), pltpu.VMEM((G, HID), f32)],
    )(agg_a, agg_b, deg, t2, b2col, Wsl, bsl.reshape(1, HID), cntc)

    out = pl.pallas_call(
        _d_body,
        out_shape=jax.ShapeDtypeStruct((G, 3), f32),
    )(gmax, gmean, W1, b1.reshape(1, -1), W2, b2.reshape(1, -1),
      W3, b3.reshape(1, -1))
    return out


# speedup vs baseline: 30.0028x; 1.2625x over previous
import functools

import jax
import jax.numpy as jnp
from jax import lax
from jax.experimental import pallas as pl
from jax.experimental.pallas import tpu as pltpu
from jax.experimental.pallas import tpu_sc as plsc

N = 50000
E = 800000
G = 16
H = 16
C = 4
HC = H * C
HID = 64

NW = 32
N_P = 50176
E_P = 819200
EPW = E_P // NW
EPC = E_P // 16
K = 128
NIT = EPW // K
NIT2 = EPC // K
SAGE_W = 32
DEG_W = 8
HALF = N_P // 2
HR = HALF + 128
RPH = HR // 16

BLK = 512
NB = N_P // BLK

_mesh = plsc.VectorSubcoreMesh(core_axis_name="c", subcore_axis_name="s")
_sc_params = pltpu.CompilerParams(needs_layout_passes=False,
                                  use_tc_tiling_on_sc=False)


def _zero_acc(zeros_hbm, acc_sh, sid):
    r0 = pl.multiple_of(sid * RPH, 8)
    pltpu.sync_copy(zeros_hbm, acc_sh.at[pl.ds(r0, RPH)])


@functools.partial(
    pl.kernel,
    out_type=(
        jax.ShapeDtypeStruct((E_P, H), jnp.float32),
        jax.ShapeDtypeStruct((E_P, H), jnp.float32),
    ),
    mesh=_mesh,
    compiler_params=_sc_params,
    scratch_types=[
        pltpu.VMEM((N_P,), jnp.float32),
        pltpu.VMEM((K,), jnp.int32),
        pltpu.VMEM((K,), jnp.int32),
        pltpu.VMEM((K,), jnp.float32),
        pltpu.VMEM((K, H), jnp.float32),
        pltpu.VMEM((K, H), jnp.float32),
        pltpu.VMEM((5 * C * 16,), jnp.float32),
    ],
)
def _gat_edge_sc(x_hbm, src_hbm, dst_hbm, ea_hbm, wp_hbm, ow_hbm, owx_hbm,
                 x_v, src_v, dst_v, ea_v, payw_v, paywx_v, wp_v):
    cid = lax.axis_index("c")
    sid = lax.axis_index("s")
    wid = sid * 2 + cid

    pltpu.sync_copy(wp_hbm, wp_v)
    pltpu.sync_copy(x_hbm, x_v)

    wv = [[wp_v[pl.ds((j * C + c) * 16, 16)] for c in range(C)]
          for j in range(5)]

    base = wid * EPW

    @pl.loop(0, NIT)
    def _chunk(it):
        off = pl.multiple_of(base + it * K, 8)
        pltpu.sync_copy(
            (src_hbm.at[pl.ds(off, K)], dst_hbm.at[pl.ds(off, K)],
             ea_hbm.at[pl.ds(off, K)]),
            (src_v, dst_v, ea_v))

        @pl.loop(0, K // 16)
        def _group(g):
            g16 = g * 16
            s16 = src_v[pl.ds(g16, 16)]
            d16 = dst_v[pl.ds(g16, 16)]
            e16 = ea_v[pl.ds(g16, 16)]
            xs16 = plsc.load_gather(x_v, [s16])
            xd16 = plsc.load_gather(x_v, [d16])
            for l in range(16):
                xs = xs16[l]
                xd = xd16[l]
                ea = e16[l]
                lg = None
                for c in range(C):
                    t = (xs * wv[0][c] + xd * wv[1][c]
                         + ea * wv[2][c] + wv[3][c])
                    m = jnp.maximum(t, 0.2 * t)
                    contrib = m * wv[4][c]
                    lg = contrib if lg is None else lg + contrib
                w = jnp.exp(lg)
                r = g16 + l
                payw_v[r] = w
                paywx_v[r] = w * xs

        pltpu.sync_copy(payw_v, ow_hbm.at[pl.ds(off, K)])
        pltpu.sync_copy(paywx_v, owx_hbm.at[pl.ds(off, K)])


@functools.partial(
    pl.kernel,
    out_type=(
        jax.ShapeDtypeStruct((2, HR, H), jnp.float32),
        jax.ShapeDtypeStruct((2, HR, H), jnp.float32),
        jax.ShapeDtypeStruct((2, HR, DEG_W), jnp.float32),
    ),
    mesh=_mesh,
    compiler_params=_sc_params,
    scratch_types=[
        pltpu.VMEM((K,), jnp.int32),
        pltpu.VMEM((K, H), jnp.float32),
        pltpu.VMEM((K, H), jnp.float32),
        pltpu.VMEM((K, DEG_W), jnp.float32),
        pltpu.VMEM_SHARED((HR, H), jnp.float32),
        pltpu.VMEM_SHARED((HR, H), jnp.float32),
        pltpu.VMEM_SHARED((HR, DEG_W), jnp.float32),
    ],
)
def _scat_sc(pw_hbm, pwx_hbm, idx_hbm, ones_hbm, z16_hbm, z8_hbm,
             ow_hbm, owx_hbm, od_hbm,
             idx_v, payw_v, paywx_v, payd_v,
             accw_sh, accwx_sh, accd_sh):
    cid = lax.axis_index("c")
    sid = lax.axis_index("s")

    pltpu.sync_copy(ones_hbm, payd_v)
    _zero_acc(z16_hbm, accw_sh, sid)
    _zero_acc(z16_hbm, accwx_sh, sid)
    _zero_acc(z8_hbm, accd_sh, sid)
    plsc.subcore_barrier()

    base = sid * EPC

    @pl.loop(0, NIT2)
    def _chunk(it):
        off = pl.multiple_of(base + it * K, 8)
        pltpu.sync_copy(
            (idx_hbm.at[cid, pl.ds(off, K)], pw_hbm.at[pl.ds(off, K)],
             pwx_hbm.at[pl.ds(off, K)]),
            (idx_v, payw_v, paywx_v))
        pltpu.sync_copy(payw_v, accw_sh.at[idx_v], add=True)
        pltpu.sync_copy(paywx_v, accwx_sh.at[idx_v], add=True)
        pltpu.sync_copy(payd_v, accd_sh.at[idx_v], add=True)

    plsc.subcore_barrier()
    r0 = pl.multiple_of(sid * RPH, 8)
    sl = pl.ds(r0, RPH)
    pltpu.sync_copy(
        (accw_sh.at[sl], accwx_sh.at[sl], accd_sh.at[sl]),
        (ow_hbm.at[cid, sl], owx_hbm.at[cid, sl], od_hbm.at[cid, sl]))


@functools.partial(
    pl.kernel,
    out_type=jax.ShapeDtypeStruct((2, HR, SAGE_W), jnp.float32),
    mesh=_mesh,
    compiler_params=_sc_params,
    scratch_types=[
        pltpu.VMEM((K,), jnp.int32),
        pltpu.VMEM((K,), jnp.int32),
        pltpu.VMEM((K, SAGE_W), jnp.float32),
        pltpu.VMEM_SHARED((HR, SAGE_W), jnp.float32),
        pltpu.SemaphoreType.DMA,
    ],
)
def _sage_sc(h1_hbm, src_hbm, idx_hbm, z32_hbm, out_hbm,
             src_v, idx_v, rows_v, acc_sh, sem):
    cid = lax.axis_index("c")
    sid = lax.axis_index("s")

    _zero_acc(z32_hbm, acc_sh, sid)
    plsc.subcore_barrier()

    base = sid * EPC

    @pl.loop(0, NIT2)
    def _chunk(it):
        off = pl.multiple_of(base + it * K, 8)
        pltpu.sync_copy(
            (src_hbm.at[pl.ds(off, K)], idx_hbm.at[cid, pl.ds(off, K)]),
            (src_v, idx_v))
        pltpu.async_copy(h1_hbm.at[src_v], rows_v, sem).wait()
        pltpu.sync_copy(rows_v, acc_sh.at[idx_v], add=True)

    plsc.subcore_barrier()
    r0 = pl.multiple_of(sid * RPH, 8)
    sl = pl.ds(r0, RPH)
    pltpu.sync_copy(acc_sh.at[sl], out_hbm.at[cid, sl])


def _easum_body(ea_ref, out_ref):
    out_ref[...] = jnp.sum(ea_ref[...]).reshape(1, 1)


def _idx_body(dst_ref, i0_ref, i1_ref):
    d = dst_ref[...]
    i0_ref[...] = jnp.where(d < HALF, d, HALF)
    i1_ref[...] = jnp.where(d >= HALF, d - HALF, HALF)


def _onehot(batch_col):
    io = lax.broadcasted_iota(jnp.int32, (1, G), 1)
    return (batch_col == io).astype(jnp.float32)


def _b1_body(accw_ref, accwx_ref, accd_ref, x_ref, b_ref, wsum_ref, we_ref,
             bsum_ref, amat_ref, wl_ref, bl_ref, bg_ref, eh_ref, easum_ref,
             h1_ref, deg_ref, sums_ref, cnt_ref, sums_s, cnt_s):
    i = pl.program_id(0)

    @pl.when(i == 0)
    def _():
        sums_s[...] = jnp.zeros_like(sums_s)
        cnt_s[...] = jnp.zeros_like(cnt_s)

    den16 = accw_ref[...]
    s116 = accwx_ref[...]
    deg_ref[...] = accd_ref[:, 0:1]

    xb = x_ref[...]
    eamean = easum_ref[0, 0] * (1.0 / E)
    t = xb * wsum_ref[...] + eamean * we_ref[...] + bsum_ref[...]
    m = jnp.maximum(t, 0.2 * t)
    logits = jnp.dot(m, amat_ref[...], preferred_element_type=jnp.float32)
    wself = jnp.exp(logits)
    den16 = den16 + wself
    s116 = s116 + wself * xb

    den64 = jnp.dot(den16, eh_ref[...], preferred_element_type=jnp.float32)
    s164 = jnp.dot(s116, eh_ref[...], preferred_element_type=jnp.float32)
    num = wl_ref[...] * s164 + bl_ref[...] * den64
    h1p = jax.nn.relu(num / (den64 + 1e-16) + bg_ref[...])
    h1_ref[...] = h1p

    oh = _onehot(b_ref[...])
    sums_s[...] += lax.dot_general(oh, h1p, (((0,), (0,)), ((), ())),
                                   preferred_element_type=jnp.float32)
    cnt_s[...] += jnp.sum(oh, axis=0, keepdims=True)

    @pl.when(i == NB - 1)
    def _():
        sums_ref[...] = sums_s[...]
        cnt_ref[...] = cnt_s[...]


def _b2_body(h1_ref, b_ref, sums_ref, cntc_ref, gnms_ref,
             cen_ref, ssq_ref, ssq_s):
    i = pl.program_id(0)

    @pl.when(i == 0)
    def _():
        ssq_s[...] = jnp.zeros_like(ssq_s)

    mean = sums_ref[...] / cntc_ref[...]
    oh = _onehot(b_ref[...])
    mb = jnp.dot(oh, mean, preferred_element_type=jnp.float32)
    cen = h1_ref[...] - gnms_ref[...] * mb
    cen_ref[...] = cen
    ssq_s[...] += lax.dot_general(oh, cen * cen, (((0,), (0,)), ((), ())),
                                  preferred_element_type=jnp.float32)

    @pl.when(i == NB - 1)
    def _():
        ssq_ref[...] = ssq_s[...]


def _b3_body(cen_ref, b_ref, ssq_ref, cntc_ref, gnw_ref, gnb_ref, wsr_ref,
             h1a_ref, h1b_ref, t2_ref):
    var = ssq_ref[...] / cntc_ref[...]
    scale = 1.0 / jnp.sqrt(var + 1e-5)
    oh = _onehot(b_ref[...])
    sb = jnp.dot(oh, scale, preferred_element_type=jnp.float32)
    h1 = gnw_ref[...] * cen_ref[...] * sb + gnb_ref[...]
    h1a_ref[...] = h1[:, :32]
    h1b_ref[...] = h1[:, 32:]
    t2_ref[...] = jnp.dot(h1, wsr_ref[...], preferred_element_type=jnp.float32)


def _c_body(aa_ref, ab_ref, deg_ref, t2_ref, b_ref, wsl_ref, bsl_ref,
            cntc_ref, gmax_ref, gmean_ref, gmax_s, gsum_s):
    i = pl.program_id(0)

    @pl.when(i == 0)
    def _():
        gmax_s[...] = jnp.full_like(gmax_s, -jnp.inf)
        gsum_s[...] = jnp.zeros_like(gsum_s)

    agg = jnp.concatenate([aa_ref[...], ab_ref[...]], axis=1)
    degc = jnp.maximum(deg_ref[...], 1.0)
    aggn = agg / degc
    h2 = jax.nn.relu(
        jnp.dot(aggn, wsl_ref[...], preferred_element_type=jnp.float32)
        + bsl_ref[...] + t2_ref[...])

    bcol = b_ref[...]
    for g in range(G):
        mask = bcol == g
        vals = jnp.where(mask, h2, -jnp.inf)
        mg = jnp.max(vals, axis=0, keepdims=True)
        gmax_s[pl.ds(g, 1), :] = jnp.maximum(gmax_s[pl.ds(g, 1), :], mg)

    oh = _onehot(bcol)
    gsum_s[...] += lax.dot_general(oh, h2, (((0,), (0,)), ((), ())),
                                   preferred_element_type=jnp.float32)

    @pl.when(i == NB - 1)
    def _():
        gm = gmax_s[...]
        gmax_ref[...] = jnp.where(jnp.isfinite(gm), gm, 0.0)
        gmean_ref[...] = gsum_s[...] / cntc_ref[...]


def _d_body(gmax_ref, gmean_ref, w1_ref, b1_ref, w2_ref, b2_ref,
            w3_ref, b3_ref, out_ref):
    z = jnp.concatenate([gmax_ref[...], gmean_ref[...]], axis=1)
    z = jax.nn.relu(jnp.dot(z, w1_ref[...], preferred_element_type=jnp.float32)
                    + b1_ref[...])
    z = jax.nn.relu(jnp.dot(z, w2_ref[...], preferred_element_type=jnp.float32)
                    + b2_ref[...])
    out_ref[...] = (jnp.dot(z, w3_ref[...], preferred_element_type=jnp.float32)
                    + b3_ref[...])


def _full(shape):
    return pl.BlockSpec(shape, lambda i: tuple(0 for _ in shape))


def _rows(width):
    return pl.BlockSpec((BLK, width), lambda i: (i, 0))


def kernel(x, edge_index, edge_attr, batch, Wl, bl, Wr, br, We, att, bg,
           Wsl, bsl, Wsr, gn_w, gn_b, gn_ms, W1, b1, W2, b2, W3, b3):
    f32 = jnp.float32
    src = edge_index[0].astype(jnp.int32)
    dst = edge_index[1].astype(jnp.int32)

    pe = E_P - E
    srcp = jnp.concatenate([src, jnp.full((pe,), N, jnp.int32)])
    dstp = jnp.concatenate([dst, jnp.full((pe,), N, jnp.int32)])
    eap = jnp.concatenate([edge_attr.astype(f32), jnp.zeros((pe,), f32)])
    xp = jnp.concatenate([x.astype(f32), jnp.zeros((N_P - N,), f32)])
    batchp = jnp.concatenate([batch.astype(jnp.int32),
                              jnp.full((N_P - N,), G, jnp.int32)])
    x2 = xp.reshape(N_P, 1)
    b2col = batchp.reshape(N_P, 1)

    wl = Wl.reshape(1, HC)
    wr = Wr.reshape(1, HC)
    we = We.reshape(1, HC)
    blr = bl.reshape(1, HC)
    bsum = (bl + br).reshape(1, HC)
    attf = att.reshape(HC)
    wpack = jnp.concatenate(
        [a.reshape(H, C).T.reshape(1, HC)
         for a in (Wl[0], Wr[0], We[0], bl + br, attf)], axis=0).reshape(-1)
    eh = jnp.kron(jnp.eye(H, dtype=f32), jnp.ones((1, C), f32))
    amat = eh.T * attf[:, None]

    easum = pl.pallas_call(
        _easum_body,
        out_shape=jax.ShapeDtypeStruct((1, 1), f32),
    )(eap.reshape(E_P // 128, 128))

    zer16 = jnp.zeros((RPH, H), f32)
    zer32 = jnp.zeros((RPH, SAGE_W), f32)
    zer8 = jnp.zeros((RPH, DEG_W), f32)
    ones8 = jnp.ones((K, DEG_W), f32)
    idx0, idx1 = pl.pallas_call(
        _idx_body,
        out_shape=[jax.ShapeDtypeStruct((E_P // 128, 128), jnp.int32)] * 2,
    )(dstp.reshape(E_P // 128, 128))
    idx2 = jnp.stack([idx0.reshape(E_P), idx1.reshape(E_P)])
    pay_w, pay_wx = _gat_edge_sc(xp, srcp, dstp, eap, wpack)
    ow, owx, od = _scat_sc(pay_w, pay_wx, idx2, ones8, zer16, zer8)
    acc_w = jnp.concatenate([ow[0, :HALF], ow[1, :HALF]], axis=0)
    acc_wx = jnp.concatenate([owx[0, :HALF], owx[1, :HALF]], axis=0)
    acc_d = jnp.concatenate([od[0, :HALF], od[1, :HALF]], axis=0)

    h1p, deg, sums, cnt = pl.pallas_call(
        _b1_body,
        grid=(NB,),
        in_specs=[
            _rows(H), _rows(H), _rows(DEG_W),
            _rows(1), _rows(1),
            _full((1, HC)), _full((1, HC)), _full((1, HC)),
            _full((HC, H)), _full((1, HC)), _full((1, HC)), _full((1, HC)),
            _full((H, HC)), _full((1, 1)),
        ],
        out_specs=[_rows(HC), _rows(1), _full((G, HC)), _full((1, G))],
        out_shape=[
            jax.ShapeDtypeStruct((N_P, HC), f32),
            jax.ShapeDtypeStruct((N_P, 1), f32),
            jax.ShapeDtypeStruct((G, HC), f32),
            jax.ShapeDtypeStruct((1, G), f32),
        ],
        scratch_shapes=[pltpu.VMEM((G, HC), f32), pltpu.VMEM((1, G), f32)],
    )(acc_w, acc_wx, acc_d, x2, b2col, wl + wr, we, bsum, amat, wl, blr,
      bg.reshape(1, HC), eh, easum)

    cntc = jnp.maximum(cnt.reshape(G, 1), 1.0)

    cen, ssq = pl.pallas_call(
        _b2_body,
        grid=(NB,),
        in_specs=[_rows(HC), _rows(1), _full((G, HC)), _full((G, 1)),
                  _full((1, HC))],
        out_specs=[_rows(HC), _full((G, HC))],
        out_shape=[jax.ShapeDtypeStruct((N_P, HC), f32),
                   jax.ShapeDtypeStruct((G, HC), f32)],
        scratch_shapes=[pltpu.VMEM((G, HC), f32)],
    )(h1p, b2col, sums, cntc, gn_ms.reshape(1, HC))

    h1a, h1b, t2 = pl.pallas_call(
        _b3_body,
        grid=(NB,),
        in_specs=[_rows(HC), _rows(1), _full((G, HC)), _full((G, 1)),
                  _full((1, HC)), _full((1, HC)), _full((HC, HID))],
        out_specs=[_rows(32), _rows(32), _rows(HID)],
        out_shape=[jax.ShapeDtypeStruct((N_P, 32), f32),
                   jax.ShapeDtypeStruct((N_P, 32), f32),
                   jax.ShapeDtypeStruct((N_P, HID), f32)],
    )(cen, b2col, ssq, cntc, gn_w.reshape(1, HC), gn_b.reshape(1, HC), Wsr)

    def _sage_full(h1half):
        o = _sage_sc(h1half, srcp, idx2, zer32)
        return jnp.concatenate([o[0, :HALF], o[1, :HALF]], axis=0)

    agg_a = _sage_full(h1a)
    agg_b = _sage_full(h1b)

    gmax, gmean = pl.pallas_call(
        _c_body,
        grid=(NB,),
        in_specs=[
            _rows(SAGE_W), _rows(SAGE_W),
            _rows(1), _rows(HID), _rows(1),
            _full((HID, HID)), _full((1, HID)), _full((G, 1)),
        ],
        out_specs=[_full((G, HID)), _full((G, HID))],
        out_shape=[jax.ShapeDtypeStruct((G, HID), f32),
                   jax.ShapeDtypeStruct((G, HID), f32)],
        scratch_shapes=[pltpu.VMEM((G, HID), f32), pltpu.VMEM((G, HID), f32)],
    )(agg_a, agg_b, deg, t2, b2col, Wsl, bsl.reshape(1, HID), cntc)

    out = pl.pallas_call(
        _d_body,
        out_shape=jax.ShapeDtypeStruct((G, 3), f32),
    )(gmax, gmean, W1, b1.reshape(1, -1), W2, b2.reshape(1, -1),
      W3, b3.reshape(1, -1))
    return out


# speedup vs baseline: 36.9198x; 1.2305x over previous
import functools

import jax
import jax.numpy as jnp
from jax import lax
from jax.experimental import pallas as pl
from jax.experimental.pallas import tpu as pltpu
from jax.experimental.pallas import tpu_sc as plsc

N = 50000
E = 800000
G = 16
H = 16
C = 4
HC = H * C
HID = 64

NW = 32
N_P = 50176
E_P = 819200
EPW = E_P // NW
EPC = E_P // 16
K = 128
NIT = EPW // K
NIT2 = EPC // K
SAGE_W = 32
PAY_W = 40
DEG_W = 8
HALF = N_P // 2
HR = HALF + 128
RPH = HR // 16

BLK = 512
NB = N_P // BLK

_mesh = plsc.VectorSubcoreMesh(core_axis_name="c", subcore_axis_name="s")
_sc_params = pltpu.CompilerParams(needs_layout_passes=False,
                                  use_tc_tiling_on_sc=False)


def _zero_acc(zeros_hbm, acc_sh, sid):
    r0 = pl.multiple_of(sid * RPH, 8)
    pltpu.sync_copy(zeros_hbm, acc_sh.at[pl.ds(r0, RPH)])


@functools.partial(
    pl.kernel,
    out_type=jax.ShapeDtypeStruct((E_P, PAY_W), jnp.float32),
    mesh=_mesh,
    compiler_params=_sc_params,
    scratch_types=[
        pltpu.VMEM((N_P,), jnp.float32),
        pltpu.VMEM((K,), jnp.int32),
        pltpu.VMEM((K,), jnp.int32),
        pltpu.VMEM((K,), jnp.float32),
        pltpu.VMEM((K, PAY_W), jnp.float32),
        pltpu.VMEM((5 * C * 16,), jnp.float32),
    ],
)
def _gat_edge_sc(x_hbm, src_hbm, dst_hbm, ea_hbm, wp_hbm, ones_hbm, pay_hbm,
                 x_v, src_v, dst_v, ea_v, pay_v, wp_v):
    cid = lax.axis_index("c")
    sid = lax.axis_index("s")
    wid = sid * 2 + cid

    pltpu.sync_copy(wp_hbm, wp_v)
    pltpu.sync_copy(x_hbm, x_v)
    pltpu.sync_copy(ones_hbm, pay_v.at[:, pl.ds(32, DEG_W)])

    wv = [[wp_v[pl.ds((j * C + c) * 16, 16)] for c in range(C)]
          for j in range(5)]

    base = wid * EPW

    @pl.loop(0, NIT)
    def _chunk(it):
        off = pl.multiple_of(base + it * K, 8)
        pltpu.sync_copy(
            (src_hbm.at[pl.ds(off, K)], dst_hbm.at[pl.ds(off, K)],
             ea_hbm.at[pl.ds(off, K)]),
            (src_v, dst_v, ea_v))

        @pl.loop(0, K // 16)
        def _group(g):
            g16 = g * 16
            s16 = src_v[pl.ds(g16, 16)]
            d16 = dst_v[pl.ds(g16, 16)]
            e16 = ea_v[pl.ds(g16, 16)]
            xs16 = plsc.load_gather(x_v, [s16])
            xd16 = plsc.load_gather(x_v, [d16])
            for l in range(16):
                xs = xs16[l]
                xd = xd16[l]
                ea = e16[l]
                lg = None
                for c in range(C):
                    t = (xs * wv[0][c] + xd * wv[1][c]
                         + ea * wv[2][c] + wv[3][c])
                    m = jnp.maximum(t, 0.2 * t)
                    contrib = m * wv[4][c]
                    lg = contrib if lg is None else lg + contrib
                w = jnp.exp(lg)
                r = g16 + l
                pay_v[r, pl.ds(0, 16)] = w
                pay_v[r, pl.ds(16, 16)] = w * xs

        pltpu.sync_copy(pay_v, pay_hbm.at[pl.ds(off, K)])


@functools.partial(
    pl.kernel,
    out_type=jax.ShapeDtypeStruct((2, HR, PAY_W), jnp.float32),
    mesh=_mesh,
    compiler_params=_sc_params,
    scratch_types=[
        pltpu.VMEM((K,), jnp.int32),
        pltpu.VMEM((K, PAY_W), jnp.float32),
        pltpu.VMEM_SHARED((HR, PAY_W), jnp.float32),
    ],
)
def _scat_sc(pay_hbm, idx_hbm, z40_hbm, out_hbm,
             idx_v, pay_v, acc_sh):
    cid = lax.axis_index("c")
    sid = lax.axis_index("s")

    _zero_acc(z40_hbm, acc_sh, sid)
    plsc.subcore_barrier()

    base = sid * EPC

    @pl.loop(0, NIT2)
    def _chunk(it):
        off = pl.multiple_of(base + it * K, 8)
        pltpu.sync_copy(
            (idx_hbm.at[cid, pl.ds(off, K)], pay_hbm.at[pl.ds(off, K)]),
            (idx_v, pay_v))
        pltpu.sync_copy(pay_v, acc_sh.at[idx_v], add=True)

    plsc.subcore_barrier()
    r0 = pl.multiple_of(sid * RPH, 8)
    sl = pl.ds(r0, RPH)
    pltpu.sync_copy(acc_sh.at[sl], out_hbm.at[cid, sl])


@functools.partial(
    pl.kernel,
    out_type=jax.ShapeDtypeStruct((2, HR, SAGE_W), jnp.float32),
    mesh=_mesh,
    compiler_params=_sc_params,
    scratch_types=[
        pltpu.VMEM((K,), jnp.int32),
        pltpu.VMEM((K,), jnp.int32),
        pltpu.VMEM((K, SAGE_W), jnp.float32),
        pltpu.VMEM_SHARED((HR, SAGE_W), jnp.float32),
        pltpu.SemaphoreType.DMA,
    ],
)
def _sage_sc(h1_hbm, src_hbm, idx_hbm, z32_hbm, out_hbm,
             src_v, idx_v, rows_v, acc_sh, sem):
    cid = lax.axis_index("c")
    sid = lax.axis_index("s")

    _zero_acc(z32_hbm, acc_sh, sid)
    plsc.subcore_barrier()

    base = sid * EPC

    @pl.loop(0, NIT2)
    def _chunk(it):
        off = pl.multiple_of(base + it * K, 8)
        pltpu.sync_copy(
            (src_hbm.at[pl.ds(off, K)], idx_hbm.at[cid, pl.ds(off, K)]),
            (src_v, idx_v))
        pltpu.async_copy(h1_hbm.at[src_v], rows_v, sem).wait()
        pltpu.sync_copy(rows_v, acc_sh.at[idx_v], add=True)

    plsc.subcore_barrier()
    r0 = pl.multiple_of(sid * RPH, 8)
    sl = pl.ds(r0, RPH)
    pltpu.sync_copy(acc_sh.at[sl], out_hbm.at[cid, sl])


def _easum_body(ea_ref, out_ref):
    out_ref[...] = jnp.sum(ea_ref[...]).reshape(1, 1)


def _idx_body(dst_ref, i0_ref, i1_ref):
    d = dst_ref[...]
    i0_ref[...] = jnp.where(d < HALF, d, HALF)
    i1_ref[...] = jnp.where(d >= HALF, d - HALF, HALF)


def _onehot(batch_col):
    io = lax.broadcasted_iota(jnp.int32, (1, G), 1)
    return (batch_col == io).astype(jnp.float32)


def _b1_body(acc_ref, x_ref, b_ref, wsum_ref, we_ref,
             bsum_ref, amat_ref, wl_ref, bl_ref, bg_ref, eh_ref, easum_ref,
             h1_ref, deg_ref, sums_ref, cnt_ref, sums_s, cnt_s):
    i = pl.program_id(0)

    @pl.when(i == 0)
    def _():
        sums_s[...] = jnp.zeros_like(sums_s)
        cnt_s[...] = jnp.zeros_like(cnt_s)

    a = acc_ref[...]
    den16 = a[:, 0:16]
    s116 = a[:, 16:32]
    deg_ref[...] = a[:, 32:33]

    xb = x_ref[...]
    eamean = easum_ref[0, 0] * (1.0 / E)
    t = xb * wsum_ref[...] + eamean * we_ref[...] + bsum_ref[...]
    m = jnp.maximum(t, 0.2 * t)
    logits = jnp.dot(m, amat_ref[...], preferred_element_type=jnp.float32)
    wself = jnp.exp(logits)
    den16 = den16 + wself
    s116 = s116 + wself * xb

    den64 = jnp.dot(den16, eh_ref[...], preferred_element_type=jnp.float32)
    s164 = jnp.dot(s116, eh_ref[...], preferred_element_type=jnp.float32)
    num = wl_ref[...] * s164 + bl_ref[...] * den64
    h1p = jax.nn.relu(num / (den64 + 1e-16) + bg_ref[...])
    h1_ref[...] = h1p

    oh = _onehot(b_ref[...])
    sums_s[...] += lax.dot_general(oh, h1p, (((0,), (0,)), ((), ())),
                                   preferred_element_type=jnp.float32)
    cnt_s[...] += jnp.sum(oh, axis=0, keepdims=True)

    @pl.when(i == NB - 1)
    def _():
        sums_ref[...] = sums_s[...]
        cnt_ref[...] = cnt_s[...]


def _b2_body(h1_ref, b_ref, sums_ref, cntc_ref, gnms_ref,
             cen_ref, ssq_ref, ssq_s):
    i = pl.program_id(0)

    @pl.when(i == 0)
    def _():
        ssq_s[...] = jnp.zeros_like(ssq_s)

    mean = sums_ref[...] / cntc_ref[...]
    oh = _onehot(b_ref[...])
    mb = jnp.dot(oh, mean, preferred_element_type=jnp.float32)
    cen = h1_ref[...] - gnms_ref[...] * mb
    cen_ref[...] = cen
    ssq_s[...] += lax.dot_general(oh, cen * cen, (((0,), (0,)), ((), ())),
                                  preferred_element_type=jnp.float32)

    @pl.when(i == NB - 1)
    def _():
        ssq_ref[...] = ssq_s[...]


def _b3_body(cen_ref, b_ref, ssq_ref, cntc_ref, gnw_ref, gnb_ref, wsr_ref,
             h1a_ref, h1b_ref, t2_ref):
    var = ssq_ref[...] / cntc_ref[...]
    scale = 1.0 / jnp.sqrt(var + 1e-5)
    oh = _onehot(b_ref[...])
    sb = jnp.dot(oh, scale, preferred_element_type=jnp.float32)
    h1 = gnw_ref[...] * cen_ref[...] * sb + gnb_ref[...]
    h1a_ref[...] = h1[:, :32]
    h1b_ref[...] = h1[:, 32:]
    t2_ref[...] = jnp.dot(h1, wsr_ref[...], preferred_element_type=jnp.float32)


def _c_body(aa_ref, ab_ref, deg_ref, t2_ref, b_ref, wsl_ref, bsl_ref,
            cntc_ref, gmax_ref, gmean_ref, gmax_s, gsum_s):
    i = pl.program_id(0)

    @pl.when(i == 0)
    def _():
        gmax_s[...] = jnp.full_like(gmax_s, -jnp.inf)
        gsum_s[...] = jnp.zeros_like(gsum_s)

    agg = jnp.concatenate([aa_ref[...], ab_ref[...]], axis=1)
    degc = jnp.maximum(deg_ref[...], 1.0)
    aggn = agg / degc
    h2 = jax.nn.relu(
        jnp.dot(aggn, wsl_ref[...], preferred_element_type=jnp.float32)
        + bsl_ref[...] + t2_ref[...])

    bcol = b_ref[...]
    for g in range(G):
        mask = bcol == g
        vals = jnp.where(mask, h2, -jnp.inf)
        mg = jnp.max(vals, axis=0, keepdims=True)
        gmax_s[pl.ds(g, 1), :] = jnp.maximum(gmax_s[pl.ds(g, 1), :], mg)

    oh = _onehot(bcol)
    gsum_s[...] += lax.dot_general(oh, h2, (((0,), (0,)), ((), ())),
                                   preferred_element_type=jnp.float32)

    @pl.when(i == NB - 1)
    def _():
        gm = gmax_s[...]
        gmax_ref[...] = jnp.where(jnp.isfinite(gm), gm, 0.0)
        gmean_ref[...] = gsum_s[...] / cntc_ref[...]


def _d_body(gmax_ref, gmean_ref, w1_ref, b1_ref, w2_ref, b2_ref,
            w3_ref, b3_ref, out_ref):
    z = jnp.concatenate([gmax_ref[...], gmean_ref[...]], axis=1)
    z = jax.nn.relu(jnp.dot(z, w1_ref[...], preferred_element_type=jnp.float32)
                    + b1_ref[...])
    z = jax.nn.relu(jnp.dot(z, w2_ref[...], preferred_element_type=jnp.float32)
                    + b2_ref[...])
    out_ref[...] = (jnp.dot(z, w3_ref[...], preferred_element_type=jnp.float32)
                    + b3_ref[...])


def _full(shape):
    return pl.BlockSpec(shape, lambda i: tuple(0 for _ in shape))


def _rows(width):
    return pl.BlockSpec((BLK, width), lambda i: (i, 0))


def kernel(x, edge_index, edge_attr, batch, Wl, bl, Wr, br, We, att, bg,
           Wsl, bsl, Wsr, gn_w, gn_b, gn_ms, W1, b1, W2, b2, W3, b3):
    f32 = jnp.float32
    src = edge_index[0].astype(jnp.int32)
    dst = edge_index[1].astype(jnp.int32)

    pe = E_P - E
    srcp = jnp.concatenate([src, jnp.full((pe,), N, jnp.int32)])
    dstp = jnp.concatenate([dst, jnp.full((pe,), N, jnp.int32)])
    eap = jnp.concatenate([edge_attr.astype(f32), jnp.zeros((pe,), f32)])
    xp = jnp.concatenate([x.astype(f32), jnp.zeros((N_P - N,), f32)])
    batchp = jnp.concatenate([batch.astype(jnp.int32),
                              jnp.full((N_P - N,), G, jnp.int32)])
    x2 = xp.reshape(N_P, 1)
    b2col = batchp.reshape(N_P, 1)

    wl = Wl.reshape(1, HC)
    wr = Wr.reshape(1, HC)
    we = We.reshape(1, HC)
    blr = bl.reshape(1, HC)
    bsum = (bl + br).reshape(1, HC)
    attf = att.reshape(HC)
    wpack = jnp.concatenate(
        [a.reshape(H, C).T.reshape(1, HC)
         for a in (Wl[0], Wr[0], We[0], bl + br, attf)], axis=0).reshape(-1)
    eh = jnp.kron(jnp.eye(H, dtype=f32), jnp.ones((1, C), f32))
    amat = eh.T * attf[:, None]

    easum = pl.pallas_call(
        _easum_body,
        out_shape=jax.ShapeDtypeStruct((1, 1), f32),
    )(eap.reshape(E_P // 128, 128))

    zer40 = jnp.zeros((RPH, PAY_W), f32)
    zer32 = jnp.zeros((RPH, SAGE_W), f32)
    ones8 = jnp.ones((K, DEG_W), f32)
    idx0, idx1 = pl.pallas_call(
        _idx_body,
        out_shape=[jax.ShapeDtypeStruct((E_P // 128, 128), jnp.int32)] * 2,
    )(dstp.reshape(E_P // 128, 128))
    idx2 = jnp.stack([idx0.reshape(E_P), idx1.reshape(E_P)])
    pay = _gat_edge_sc(xp, srcp, dstp, eap, wpack, ones8)
    oacc = _scat_sc(pay, idx2, zer40)
    acc = jnp.concatenate([oacc[0, :HALF], oacc[1, :HALF]], axis=0)

    h1p, deg, sums, cnt = pl.pallas_call(
        _b1_body,
        grid=(NB,),
        in_specs=[
            _rows(PAY_W),
            _rows(1), _rows(1),
            _full((1, HC)), _full((1, HC)), _full((1, HC)),
            _full((HC, H)), _full((1, HC)), _full((1, HC)), _full((1, HC)),
            _full((H, HC)), _full((1, 1)),
        ],
        out_specs=[_rows(HC), _rows(1), _full((G, HC)), _full((1, G))],
        out_shape=[
            jax.ShapeDtypeStruct((N_P, HC), f32),
            jax.ShapeDtypeStruct((N_P, 1), f32),
            jax.ShapeDtypeStruct((G, HC), f32),
            jax.ShapeDtypeStruct((1, G), f32),
        ],
        scratch_shapes=[pltpu.VMEM((G, HC), f32), pltpu.VMEM((1, G), f32)],
    )(acc, x2, b2col, wl + wr, we, bsum, amat, wl, blr,
      bg.reshape(1, HC), eh, easum)

    cntc = jnp.maximum(cnt.reshape(G, 1), 1.0)

    cen, ssq = pl.pallas_call(
        _b2_body,
        grid=(NB,),
        in_specs=[_rows(HC), _rows(1), _full((G, HC)), _full((G, 1)),
                  _full((1, HC))],
        out_specs=[_rows(HC), _full((G, HC))],
        out_shape=[jax.ShapeDtypeStruct((N_P, HC), f32),
                   jax.ShapeDtypeStruct((G, HC), f32)],
        scratch_shapes=[pltpu.VMEM((G, HC), f32)],
    )(h1p, b2col, sums, cntc, gn_ms.reshape(1, HC))

    h1a, h1b, t2 = pl.pallas_call(
        _b3_body,
        grid=(NB,),
        in_specs=[_rows(HC), _rows(1), _full((G, HC)), _full((G, 1)),
                  _full((1, HC)), _full((1, HC)), _full((HC, HID))],
        out_specs=[_rows(32), _rows(32), _rows(HID)],
        out_shape=[jax.ShapeDtypeStruct((N_P, 32), f32),
                   jax.ShapeDtypeStruct((N_P, 32), f32),
                   jax.ShapeDtypeStruct((N_P, HID), f32)],
    )(cen, b2col, ssq, cntc, gn_w.reshape(1, HC), gn_b.reshape(1, HC), Wsr)

    def _sage_full(h1half):
        o = _sage_sc(h1half, srcp, idx2, zer32)
        return jnp.concatenate([o[0, :HALF], o[1, :HALF]], axis=0)

    agg_a = _sage_full(h1a)
    agg_b = _sage_full(h1b)

    gmax, gmean = pl.pallas_call(
        _c_body,
        grid=(NB,),
        in_specs=[
            _rows(SAGE_W), _rows(SAGE_W),
            _rows(1), _rows(HID), _rows(1),
            _full((HID, HID)), _full((1, HID)), _full((G, 1)),
        ],
        out_specs=[_full((G, HID)), _full((G, HID))],
        out_shape=[jax.ShapeDtypeStruct((G, HID), f32),
                   jax.ShapeDtypeStruct((G, HID), f32)],
        scratch_shapes=[pltpu.VMEM((G, HID), f32), pltpu.VMEM((G, HID), f32)],
    )(agg_a, agg_b, deg, t2, b2col, Wsl, bsl.reshape(1, HID), cntc)

    out = pl.pallas_call(
        _d_body,
        out_shape=jax.ShapeDtypeStruct((G, 3), f32),
    )(gmax, gmean, W1, b1.reshape(1, -1), W2, b2.reshape(1, -1),
      W3, b3.reshape(1, -1))
    return out
